# baseline (speedup 1.0000x reference)
import functools

import jax
import jax.numpy as jnp
from jax import lax
from jax.experimental import pallas as pl
from jax.experimental.pallas import tpu as pltpu
from jax.experimental.pallas import tpu_sc as plsc

_B, _S, _D = 4, 8192, 1024
_N = 128
_ROWS = _B * _N
_NC, _NS = 2, 16
_NW = _NC * _NS
_RPW = _ROWS // _NW
_WPB = _N // _RPW


@functools.partial(
    pl.kernel,
    mesh=plsc.VectorSubcoreMesh(core_axis_name="c", subcore_axis_name="s"),
    out_type=jax.ShapeDtypeStruct((_ROWS, _D), jnp.float32),
    scratch_types=[
        pltpu.VMEM((_RPW,), jnp.int32),
        pltpu.VMEM((_RPW, _D), jnp.float32),
        pltpu.SemaphoreType.DMA,
    ],
)
def _sc_gather(x_hbm, idx_hbm, out_hbm, idx_v, rows_v, sem):
    wid = lax.axis_index("s") * _NC + lax.axis_index("c")
    batch = wid // _WPB
    pltpu.sync_copy(idx_hbm.at[pl.ds((wid % _WPB) * _RPW, _RPW)], idx_v)
    idx_v[...] = idx_v[...] + batch * _S
    pltpu.async_copy(x_hbm.at[idx_v], rows_v, sem).wait()
    pltpu.sync_copy(rows_v, out_hbm.at[pl.ds(wid * _RPW, _RPW)])


def kernel(x, index):
    x_flat = x.reshape(_B * _S, _D)
    out = _sc_gather(x_flat, index.astype(jnp.int32))
    return out.reshape(_B, _N, _D)


# speedup vs baseline: 1.0039x; 1.0039x over previous
import functools

import jax
import jax.numpy as jnp
from jax import lax
from jax.experimental import pallas as pl
from jax.experimental.pallas import tpu as pltpu
from jax.experimental.pallas import tpu_sc as plsc

_B, _S, _D = 4, 8192, 1024
_N = 128
_ROWS = _B * _N
_NC, _NS = 2, 16
_NW = _NC * _NS
_RPW = _ROWS // _NW
_WPB = _N // _RPW


_NCHUNK = 2
_CROWS = _RPW // _NCHUNK


@functools.partial(
    pl.kernel,
    mesh=plsc.VectorSubcoreMesh(core_axis_name="c", subcore_axis_name="s"),
    out_type=jax.ShapeDtypeStruct((_ROWS, _D), jnp.float32),
    scratch_types=[
        pltpu.VMEM((_RPW,), jnp.int32),
        pltpu.VMEM((_RPW, _D), jnp.float32),
        pltpu.SemaphoreType.DMA,
        pltpu.SemaphoreType.DMA,
    ],
)
def _sc_gather(x_hbm, idx_hbm, out_hbm, idx_v, rows_v, gsem, ssem):
    wid = lax.axis_index("s") * _NC + lax.axis_index("c")
    batch = wid // _WPB
    pltpu.sync_copy(idx_hbm.at[pl.ds((wid % _WPB) * _RPW, _RPW)], idx_v)
    idx_v[...] = idx_v[...] + batch * _S
    gathers = []
    for i in range(_NCHUNK):
        gathers.append(
            pltpu.async_copy(
                x_hbm.at[idx_v.at[pl.ds(i * _CROWS, _CROWS)]],
                rows_v.at[pl.ds(i * _CROWS, _CROWS)],
                gsem,
            )
        )
    stores = []
    for i in range(_NCHUNK):
        gathers[i].wait()
        stores.append(
            pltpu.async_copy(
                rows_v.at[pl.ds(i * _CROWS, _CROWS)],
                out_hbm.at[pl.ds(wid * _RPW + i * _CROWS, _CROWS)],
                ssem,
            )
        )
    for s in stores:
        s.wait()


def kernel(x, index):
    x_flat = x.reshape(_B * _S, _D)
    out = _sc_gather(x_flat, index.astype(jnp.int32))
    return out.reshape(_B, _N, _D)


# speedup vs baseline: 1.0285x; 1.0246x over previous
import functools

import jax
import jax.numpy as jnp
from jax import lax
from jax.experimental import pallas as pl
from jax.experimental.pallas import tpu as pltpu
from jax.experimental.pallas import tpu_sc as plsc

_B, _S, _D = 4, 8192, 1024
_N = 128
_ROWS = _B * _N
_NC, _NS = 1, 16
_NW = _NC * _NS
_RPW = _ROWS // _NW
_WPB = _N // _RPW


_NCHUNK = 2
_CROWS = _RPW // _NCHUNK


@functools.partial(
    pl.kernel,
    mesh=plsc.VectorSubcoreMesh(
        core_axis_name="c", subcore_axis_name="s", num_cores=1
    ),
    out_type=jax.ShapeDtypeStruct((_ROWS, _D), jnp.float32),
    scratch_types=[
        pltpu.VMEM((_RPW,), jnp.int32),
        pltpu.VMEM((_RPW, _D), jnp.float32),
        pltpu.SemaphoreType.DMA,
        pltpu.SemaphoreType.DMA,
    ],
)
def _sc_gather(x_hbm, idx_hbm, out_hbm, idx_v, rows_v, gsem, ssem):
    wid = lax.axis_index("s") * _NC + lax.axis_index("c")
    batch = wid // _WPB
    pltpu.sync_copy(idx_hbm.at[pl.ds((wid % _WPB) * _RPW, _RPW)], idx_v)
    idx_v[...] = idx_v[...] + batch * _S
    gathers = []
    for i in range(_NCHUNK):
        gathers.append(
            pltpu.async_copy(
                x_hbm.at[idx_v.at[pl.ds(i * _CROWS, _CROWS)]],
                rows_v.at[pl.ds(i * _CROWS, _CROWS)],
                gsem,
            )
        )
    stores = []
    for i in range(_NCHUNK):
        gathers[i].wait()
        stores.append(
            pltpu.async_copy(
                rows_v.at[pl.ds(i * _CROWS, _CROWS)],
                out_hbm.at[pl.ds(wid * _RPW + i * _CROWS, _CROWS)],
                ssem,
            )
        )
    for s in stores:
        s.wait()


def kernel(x, index):
    x_flat = x.reshape(_B * _S, _D)
    out = _sc_gather(x_flat, index.astype(jnp.int32))
    return out.reshape(_B, _N, _D)
